# baseline (device time: 78321 ns/iter reference)
import jax
import jax.numpy as jnp
from jax import lax
from jax.experimental import pallas as pl
from jax.experimental.pallas import tpu as pltpu

N_DEV = 32
HL = 4
DH = 64
B = 2
SQ = 256
SKV = 256
DMODEL = 512
ROWS = B * SQ
C = ROWS // N_DEV


def _body(q_ref, k_ref, v_ref, wo_ref, out_ref,
          p_ref, rs_ref, red_ref, s1, r1, s2, r2):
    my = lax.axis_index("i")

    bar = pltpu.get_barrier_semaphore()
    for j in range(N_DEV):
        @pl.when(j != my)
        def _():
            pl.semaphore_signal(
                bar, inc=1, device_id=j,
                device_id_type=pl.DeviceIdType.LOGICAL,
            )
    pl.semaphore_wait(bar, N_DEV - 1)

    ri = lax.broadcasted_iota(jnp.int32, (SQ, SKV), 0) // 64
    ci = lax.broadcasted_iota(jnp.int32, (SQ, SKV), 1) // 64
    mask = (ri == ci) | (ci == 0) | (((ri + ci) % 3) == 0)

    for b in range(B):
        acc = None
        for h in range(HL):
            qh = q_ref[b * SQ:(b + 1) * SQ, h * DH:(h + 1) * DH].astype(jnp.bfloat16)
            kh = k_ref[b, h].astype(jnp.bfloat16)
            s = lax.dot_general(
                qh, kh, (((1,), (1,)), ((), ())),
                preferred_element_type=jnp.float32,
            ) * 0.125
            s = jnp.where(mask, s, -1e9)
            m = jnp.max(s, axis=1, keepdims=True)
            w = jnp.exp(s - m)
            w = w / jnp.sum(w, axis=1, keepdims=True)
            vh = v_ref[b, h].astype(jnp.bfloat16)
            ctx = jnp.dot(w.astype(jnp.bfloat16), vh,
                          preferred_element_type=jnp.float32)
            woh = wo_ref[h * DH:(h + 1) * DH, :].astype(jnp.bfloat16)
            pb = jnp.dot(ctx.astype(jnp.bfloat16), woh,
                         preferred_element_type=jnp.float32)
            acc = pb if acc is None else acc + pb
        p_ref[b * SQ:(b + 1) * SQ, :] = acc

    for j in range(N_DEV):
        @pl.when(j != my)
        def _():
            rdma = pltpu.make_async_remote_copy(
                src_ref=p_ref.at[pl.ds(j * C, C), :],
                dst_ref=rs_ref.at[pl.ds(my * C, C), :],
                send_sem=s1.at[j],
                recv_sem=r1.at[my],
                device_id=j,
                device_id_type=pl.DeviceIdType.LOGICAL,
            )
            rdma.start()

    rs_ref[pl.ds(my * C, C), :] = p_ref[pl.ds(my * C, C), :]

    for j in range(N_DEV):
        @pl.when(j != my)
        def _():
            rd = pltpu.make_async_remote_copy(
                src_ref=p_ref.at[pl.ds(0, C), :],
                dst_ref=rs_ref.at[pl.ds(j * C, C), :],
                send_sem=s1.at[j],
                recv_sem=r1.at[j],
                device_id=j,
                device_id_type=pl.DeviceIdType.LOGICAL,
            )
            rd.wait_recv()

    red = rs_ref[0:C, :]
    for j in range(1, N_DEV):
        red = red + rs_ref[j * C:(j + 1) * C, :]
    red_ref[:, :] = red
    out_ref[pl.ds(my * C, C), :] = red

    for j in range(N_DEV):
        @pl.when(j != my)
        def _():
            rd = pltpu.make_async_remote_copy(
                src_ref=p_ref.at[pl.ds(j * C, C), :],
                dst_ref=rs_ref.at[pl.ds(j * C, C), :],
                send_sem=s1.at[j],
                recv_sem=r1.at[j],
                device_id=j,
                device_id_type=pl.DeviceIdType.LOGICAL,
            )
            rd.wait_send()

    for k in range(N_DEV):
        @pl.when(k != my)
        def _():
            rdma = pltpu.make_async_remote_copy(
                src_ref=red_ref.at[:, :],
                dst_ref=out_ref.at[pl.ds(my * C, C), :],
                send_sem=s2.at[k],
                recv_sem=r2.at[my],
                device_id=k,
                device_id_type=pl.DeviceIdType.LOGICAL,
            )
            rdma.start()

    for k in range(N_DEV):
        @pl.when(k != my)
        def _():
            rd = pltpu.make_async_remote_copy(
                src_ref=red_ref.at[:, :],
                dst_ref=out_ref.at[pl.ds(k * C, C), :],
                send_sem=s2.at[k],
                recv_sem=r2.at[k],
                device_id=k,
                device_id_type=pl.DeviceIdType.LOGICAL,
            )
            rd.wait_recv()

    for k in range(N_DEV):
        @pl.when(k != my)
        def _():
            rd = pltpu.make_async_remote_copy(
                src_ref=red_ref.at[:, :],
                dst_ref=out_ref.at[pl.ds(my * C, C), :],
                send_sem=s2.at[k],
                recv_sem=r2.at[k],
                device_id=k,
                device_id_type=pl.DeviceIdType.LOGICAL,
            )
            rd.wait_send()


def kernel(x, Wq, K_ext, V_ext, Wo):
    my = lax.axis_index("i")
    K_loc = jnp.moveaxis(
        lax.dynamic_slice_in_dim(K_ext, my * HL, HL, axis=2), 2, 1
    )
    V_loc = jnp.moveaxis(
        lax.dynamic_slice_in_dim(V_ext, my * HL, HL, axis=2), 2, 1
    )
    q = jnp.dot(
        x.reshape(ROWS, DMODEL).astype(jnp.bfloat16),
        Wq.astype(jnp.bfloat16),
        preferred_element_type=jnp.float32,
    )

    out = pl.pallas_call(
        _body,
        out_shape=jax.ShapeDtypeStruct((ROWS, DMODEL), jnp.float32),
        in_specs=[
            pl.BlockSpec(memory_space=pltpu.VMEM),
            pl.BlockSpec(memory_space=pltpu.VMEM),
            pl.BlockSpec(memory_space=pltpu.VMEM),
            pl.BlockSpec(memory_space=pltpu.VMEM),
        ],
        out_specs=pl.BlockSpec(memory_space=pltpu.VMEM),
        scratch_shapes=[
            pltpu.VMEM((ROWS, DMODEL), jnp.float32),
            pltpu.VMEM((ROWS, DMODEL), jnp.float32),
            pltpu.VMEM((C, DMODEL), jnp.float32),
            pltpu.SemaphoreType.DMA((N_DEV,)),
            pltpu.SemaphoreType.DMA((N_DEV,)),
            pltpu.SemaphoreType.DMA((N_DEV,)),
            pltpu.SemaphoreType.DMA((N_DEV,)),
        ],
        compiler_params=pltpu.CompilerParams(collective_id=0),
    )(q, K_loc, V_loc, Wo)
    return out.reshape(B, SQ, DMODEL)
